# baseline (speedup 1.0000x reference)
import functools

import jax
import jax.numpy as jnp
from jax import lax
from jax.experimental import pallas as pl
from jax.experimental.pallas import tpu as pltpu

N = 4096
NG = 8
KNN = 16
NPAD2 = 2304
RT = 256
BIGSLOT = 1 << 26
_INTERPRET = False


def _silu(x):
    return x * (1.0 / (1.0 + jnp.exp(-x)))


def _bsplit(v):
    hi = v.astype(jnp.bfloat16).astype(jnp.float32)
    lo = (v - hi).astype(jnp.bfloat16).astype(jnp.float32)
    return hi, lo


def _dot3(xi, yi, zi, xj, yj, zj):
    xih, xil = _bsplit(xi)
    yih, yil = _bsplit(yi)
    zih, zil = _bsplit(zi)
    xjh, xjl = _bsplit(xj)
    yjh, yjl = _bsplit(yj)
    zjh, zjl = _bsplit(zj)
    acc = xil * xjh
    acc = acc + yil * yjh
    acc = acc + zil * zjh
    acc = acc + xih * xjl
    acc = acc + yih * yjl
    acc = acc + zih * zjl
    acc = acc + xih * xjh
    acc = acc + yih * yjh
    acc = acc + zih * zjh
    return acc


def _fps_body(smax_ref, meta_ref, px_ref, py_ref, pz_ref, slot_ref):
    shape = (NG, N)
    lidx = lax.broadcasted_iota(jnp.int32, shape, 1)
    meta = meta_ref[...]
    ncol = meta[:, 0:1]
    mcol = meta[:, 1:2]
    offcol = meta[:, 2:3]
    valid = lidx < ncol
    x = px_ref[...]
    y = py_ref[...]
    z = pz_ref[...]

    dx = x - x[:, 0:1]
    dy = y - y[:, 0:1]
    dz = z - z[:, 0:1]
    d0 = jnp.where(valid, dx * dx + dy * dy + dz * dz, -1.0)
    slot_ref[...] = jnp.where((lidx == 0) & (mcol > 0), offcol,
                              jnp.int32(BIGSLOT))

    def body(t, d):
        active = t < mcol
        rowmax = jnp.max(d, axis=1, keepdims=True)
        nxt = jnp.min(jnp.where(d == rowmax, lidx, N), axis=1, keepdims=True)
        oh = lidx == nxt
        xs = jnp.sum(jnp.where(oh, x, 0.0), axis=1, keepdims=True)
        ys = jnp.sum(jnp.where(oh, y, 0.0), axis=1, keepdims=True)
        zs = jnp.sum(jnp.where(oh, z, 0.0), axis=1, keepdims=True)
        ex = x - xs
        ey = y - ys
        ez = z - zs
        dn = ex * ex + ey * ey + ez * ez
        d = jnp.where(active & valid, jnp.minimum(d, dn), d)
        slot_ref[...] = jnp.where(oh & active, offcol + t, slot_ref[...])
        return d

    lax.fori_loop(1, smax_ref[0, 0], body, d0)


def _run_fps(smax, meta, ppx, ppy, ppz):
    return pl.pallas_call(
        _fps_body,
        in_specs=[
            pl.BlockSpec(memory_space=pltpu.SMEM),
            pl.BlockSpec((NG, 4)),
            pl.BlockSpec((NG, N)),
            pl.BlockSpec((NG, N)),
            pl.BlockSpec((NG, N)),
        ],
        out_specs=pl.BlockSpec((NG, N)),
        out_shape=jax.ShapeDtypeStruct((NG, N), jnp.int32),
        interpret=_INTERPRET,
    )(smax, meta, ppx, ppy, ppz)


def _compact_body(slot_ref, p16_ref, out_ref):
    i = pl.program_id(0)
    rowid = i * RT + lax.broadcasted_iota(jnp.int32, (RT, 1), 0)
    oh = (slot_ref[...] == rowid).astype(jnp.float32)
    out_ref[...] = jnp.dot(oh, p16_ref[...], preferred_element_type=jnp.float32)


def _run_compact(slot_row, p16):
    return pl.pallas_call(
        _compact_body,
        grid=(NPAD2 // RT,),
        in_specs=[
            pl.BlockSpec((1, N), lambda i: (0, 0)),
            pl.BlockSpec((N, 16), lambda i: (0, 0)),
        ],
        out_specs=pl.BlockSpec((RT, 16), lambda i: (i, 0)),
        out_shape=jax.ShapeDtypeStruct((NPAD2, 16), jnp.float32),
        interpret=_INTERPRET,
    )(slot_row, p16)


def _knn_conv(d, a_full, ncols):
    citer = lax.broadcasted_iota(jnp.int32, d.shape, 1)
    m = jnp.full((d.shape[0], a_full.shape[1]), -jnp.inf, jnp.float32)
    for _ in range(KNN):
        cur = jnp.min(d, axis=1, keepdims=True)
        idx = jnp.min(jnp.where(d == cur, citer, ncols), axis=1, keepdims=True)
        sel = citer == idx
        oh = sel.astype(jnp.float32)
        gat = jnp.dot(oh, a_full, preferred_element_type=jnp.float32)
        m = jnp.maximum(m, gat)
        d = jnp.where(sel, jnp.inf, d)
    return m


def _stage1_body(p16c_ref, p8t_ref, p16_ref, btrow_ref, bc_ref,
                 w1sum_ref, w1b_ref, b1_ref, h_ref):
    rows = p16c_ref[...]
    g = _dot3(rows[:, 0:1], rows[:, 1:2], rows[:, 2:3],
              p8t_ref[0:1, :], p8t_ref[1:2, :], p8t_ref[2:3, :])
    sqi = rows[:, 3:4]
    sqj = p8t_ref[3:4, :]
    d = (sqi + sqj) - 2.0 * g
    d = jnp.where(btrow_ref[...] != bc_ref[...], jnp.inf, d)

    a1 = jnp.dot(p16_ref[...][:, :8], w1sum_ref[...],
                 preferred_element_type=jnp.float32)
    m = _knn_conv(d, a1, N)
    c = jnp.dot(rows[:, :8], w1b_ref[...],
                preferred_element_type=jnp.float32) - b1_ref[...]
    h_ref[...] = _silu(m - c)


def _run_stage1(p16c, p8t, p16, btrow, bc_col, w1sum8, w1b8, b1r):
    return pl.pallas_call(
        _stage1_body,
        grid=(NPAD2 // RT,),
        in_specs=[
            pl.BlockSpec((RT, 16), lambda i: (i, 0)),
            pl.BlockSpec((8, N), lambda i: (0, 0)),
            pl.BlockSpec((N, 16), lambda i: (0, 0)),
            pl.BlockSpec((1, N), lambda i: (0, 0)),
            pl.BlockSpec((RT, 1), lambda i: (i, 0)),
            pl.BlockSpec((8, 32), lambda i: (0, 0)),
            pl.BlockSpec((8, 32), lambda i: (0, 0)),
            pl.BlockSpec((1, 32), lambda i: (0, 0)),
        ],
        out_specs=pl.BlockSpec((RT, 32), lambda i: (i, 0)),
        out_shape=jax.ShapeDtypeStruct((NPAD2, 32), jnp.float32),
        interpret=_INTERPRET,
    )(p16c, p8t, p16, btrow, bc_col, w1sum8, w1b8, b1r)


def _stage2_body(p16c_ref, p2t_ref, h_ref, bcrow_ref, bc_ref, p16cfull_ref,
                 w2a_ref, w2b_ref, b2_ref, w3_ref, b3_ref, out_ref, gacc):
    i = pl.program_id(0)

    @pl.when(i == 0)
    def _():
        gacc[...] = jnp.full((8, 32), -jnp.inf, jnp.float32)

    rows = p16c_ref[...]
    g = _dot3(rows[:, 0:1], rows[:, 1:2], rows[:, 2:3],
              p2t_ref[0:1, :], p2t_ref[1:2, :], p2t_ref[2:3, :])
    sqi = rows[:, 3:4]
    sqj = p2t_ref[3:4, :]
    d = (sqi + sqj) - 2.0 * g
    d = jnp.where(bcrow_ref[...] != bc_ref[...], jnp.inf, d)

    a2 = (jnp.dot(h_ref[...], w2a_ref[...], preferred_element_type=jnp.float32)
          + jnp.dot(p16cfull_ref[...], w2b_ref[...],
                    preferred_element_type=jnp.float32))
    m = _knn_conv(d, a2, NPAD2)
    c = jnp.dot(rows, w2b_ref[...],
                preferred_element_type=jnp.float32) - b2_ref[...]
    h2 = _silu(m - c)

    bt = bc_ref[...]
    for gg in range(NG):
        red = jnp.max(jnp.where(bt == gg, h2, -jnp.inf), axis=0)
        gacc[gg, :] = jnp.maximum(gacc[gg, :], red)

    @pl.when(i == pl.num_programs(0) - 1)
    def _():
        out_ref[...] = (jnp.dot(gacc[...], w3_ref[...],
                                preferred_element_type=jnp.float32)
                        + b3_ref[...])


def kernel(pos, batch, W1, b1, W2, b2, W3, b3):
    pos = pos.astype(jnp.float32)
    batch = batch.astype(jnp.int32)
    sq = jnp.sum(pos * pos, axis=-1)

    starts = jnp.searchsorted(batch, jnp.arange(NG + 1, dtype=jnp.int32)
                              ).astype(jnp.int32)
    n = starts[1:] - starts[:-1]
    m = jnp.where(
        n > 0,
        jnp.maximum(1, jnp.ceil(0.5 * n.astype(jnp.float32)).astype(jnp.int32)),
        0,
    )
    off = jnp.concatenate([jnp.zeros(1, jnp.int32),
                           jnp.cumsum(m).astype(jnp.int32)])

    ranks = jnp.arange(N, dtype=jnp.int32)
    pidx_pack = jnp.clip(starts[:NG, None] + ranks[None, :], 0, N - 1)
    ppx = jnp.take(pos[:, 0], pidx_pack, axis=0)
    ppy = jnp.take(pos[:, 1], pidx_pack, axis=0)
    ppz = jnp.take(pos[:, 2], pidx_pack, axis=0)
    meta = jnp.stack([n, m, off[:NG], jnp.zeros(NG, jnp.int32)],
                     axis=1).astype(jnp.int32)
    smax = jnp.max(m).reshape(1, 1).astype(jnp.int32)

    slot_p = _run_fps(smax, meta, ppx, ppy, ppz)
    rank_v = ranks - starts[batch]
    slot_row = slot_p.reshape(-1)[batch * N + rank_v].reshape(1, N)

    p16 = jnp.concatenate(
        [pos, sq[:, None], jnp.zeros((N, 12), jnp.float32)], axis=1)
    p16c = _run_compact(slot_row, p16)

    pidx = jnp.arange(NPAD2, dtype=jnp.int32)
    batch_c = jnp.sum(pidx[:, None] >= off[None, 1:], axis=1).astype(jnp.int32)

    p8t = p16[:, :8].T
    btrow = batch.reshape(1, N)
    bc_col = batch_c.reshape(NPAD2, 1)

    w1sum8 = jnp.concatenate([W1[:3] + W1[3:], jnp.zeros((5, 32))], axis=0)
    w1b8 = jnp.concatenate([W1[3:], jnp.zeros((5, 32))], axis=0)
    h = _run_stage1(p16c, p8t, p16, btrow, bc_col,
                    w1sum8.astype(jnp.float32), w1b8.astype(jnp.float32),
                    b1.reshape(1, 32).astype(jnp.float32))

    p2t = p16c.T
    bcrow = batch_c.reshape(1, NPAD2)
    w2a = W2[:32].astype(jnp.float32)
    w2b16 = jnp.concatenate([W2[32:], jnp.zeros((13, 32))], axis=0
                            ).astype(jnp.float32)

    out = pl.pallas_call(
        _stage2_body,
        grid=(NPAD2 // RT,),
        in_specs=[
            pl.BlockSpec((RT, 16), lambda i: (i, 0)),
            pl.BlockSpec((16, NPAD2), lambda i: (0, 0)),
            pl.BlockSpec((NPAD2, 32), lambda i: (0, 0)),
            pl.BlockSpec((1, NPAD2), lambda i: (0, 0)),
            pl.BlockSpec((RT, 1), lambda i: (i, 0)),
            pl.BlockSpec((NPAD2, 16), lambda i: (0, 0)),
            pl.BlockSpec((32, 32), lambda i: (0, 0)),
            pl.BlockSpec((16, 32), lambda i: (0, 0)),
            pl.BlockSpec((1, 32), lambda i: (0, 0)),
            pl.BlockSpec((32, 128), lambda i: (0, 0)),
            pl.BlockSpec((1, 128), lambda i: (0, 0)),
        ],
        out_specs=pl.BlockSpec((NG, 128), lambda i: (0, 0)),
        out_shape=jax.ShapeDtypeStruct((NG, 128), jnp.float32),
        scratch_shapes=[pltpu.VMEM((8, 32), jnp.float32)],
        interpret=_INTERPRET,
    )(p16c, p2t, h, bcrow, bc_col, p16c,
      w2a, w2b16, b2.reshape(1, 32).astype(jnp.float32),
      W3.astype(jnp.float32), b3.reshape(1, 128).astype(jnp.float32))
    return out
